# baseline (speedup 1.0000x reference)
import functools

import jax
import jax.numpy as jnp
from jax import lax
from jax.experimental import pallas as pl
from jax.experimental.pallas import tpu as pltpu
from jax.experimental.pallas import tpu_sc as plsc

NC = 2
NS = 16
NW = NC * NS
L = 16

ENQ = 32
HALF = 512


def _gather_concat(table, idx2, B, D):
    n_per_w = idx2.shape[1]
    n_halves = n_per_w // HALF

    @functools.partial(
        pl.kernel,
        mesh=plsc.VectorSubcoreMesh(core_axis_name="c", subcore_axis_name="s"),
        out_type=jax.ShapeDtypeStruct((B, 2 * D), jnp.float32),
        scratch_types=[
            pltpu.VMEM((n_per_w,), jnp.int32),
            pltpu.VMEM((HALF, D), jnp.float32),
            pltpu.VMEM((HALF // 2, 2 * D), jnp.float32),
            pltpu.SemaphoreType.DMA,
            pltpu.SemaphoreType.DMA,
            pltpu.SemaphoreType.DMA,
        ],
        compiler_params=pltpu.CompilerParams(use_tc_tiling_on_sc=False),
    )
    def k(tab_hbm, idx_hbm, out_hbm, idx_v, rows_v, out_v, sem_i, sem_g, sem_w):
        wid = lax.axis_index("s") * NC + lax.axis_index("c")
        pltpu.async_copy(idx_hbm.at[wid], idx_v, sem_i).wait()

        for h in range(n_halves):
            def enq(step, _, h=h):
                base = h * HALF + step * ENQ
                for g in range(ENQ // L):
                    v = idx_v[pl.ds(base + g * L, L)]
                    for jj in range(L):
                        slot = step * ENQ + g * L + jj
                        pltpu.async_copy(
                            tab_hbm.at[pl.ds(v[jj], 1), :],
                            rows_v.at[pl.ds(slot, 1), :],
                            sem_g,
                        )
                return _

            lax.fori_loop(0, HALF // ENQ, enq, 0)
            pltpu.make_async_copy(
                tab_hbm.at[pl.ds(0, HALF), :], rows_v, sem_g
            ).wait()

            if h > 0:
                pltpu.make_async_copy(
                    out_v, out_hbm.at[pl.ds(0, HALF // 2)], sem_w
                ).wait()

            def pack(step, _):
                for s in range(L):
                    r = step * L + s
                    for p in range(2):
                        for q in range(D // L):
                            x = rows_v[2 * r + p, pl.ds(q * L, L)]
                            out_v[r, pl.ds(p * D + q * L, L)] = x
                return _

            lax.fori_loop(0, HALF // 2 // L, pack, 0)

            pltpu.async_copy(
                out_v,
                out_hbm.at[pl.ds(wid * (n_per_w // 2) + h * (HALF // 2),
                                 HALF // 2)],
                sem_w,
            )
        pltpu.make_async_copy(
            out_v, out_hbm.at[pl.ds(0, HALF // 2)], sem_w
        ).wait()

    return k(table, idx2)


def kernel(obj_category, sub_category, word_embs):
    (B,) = obj_category.shape
    V, D = word_embs.shape
    idx = jnp.stack(
        [obj_category.astype(jnp.int32), sub_category.astype(jnp.int32)], axis=1
    ).reshape(2 * B)
    return _gather_concat(word_embs, idx.reshape(NW, 2 * B // NW), B, D)


# speedup vs baseline: 1.6019x; 1.6019x over previous
import functools

import jax
import jax.numpy as jnp
from jax import lax
from jax.experimental import pallas as pl
from jax.experimental.pallas import tpu as pltpu
from jax.experimental.pallas import tpu_sc as plsc

NC = 2
NS = 16
NW = NC * NS
L = 16

ENQ = 32
HALF = 512


def _gather_concat(table, idx2, B, D):
    n_per_w = idx2.shape[1]
    n_halves = n_per_w // HALF

    @functools.partial(
        pl.kernel,
        mesh=plsc.VectorSubcoreMesh(core_axis_name="c", subcore_axis_name="s"),
        out_type=jax.ShapeDtypeStruct((B, 2 * D), jnp.float32),
        scratch_types=[
            pltpu.VMEM((n_per_w,), jnp.int32),
            pltpu.VMEM((HALF, D), jnp.float32),
            pltpu.VMEM((HALF // 2, 2 * D), jnp.float32),
            pltpu.SemaphoreType.DMA,
            pltpu.SemaphoreType.DMA,
            pltpu.SemaphoreType.DMA,
        ],
        compiler_params=pltpu.CompilerParams(use_tc_tiling_on_sc=True),
    )
    def k(tab_hbm, idx_hbm, out_hbm, idx_v, rows_v, out_v, sem_i, sem_g, sem_w):
        wid = lax.axis_index("s") * NC + lax.axis_index("c")
        pltpu.async_copy(idx_hbm.at[wid], idx_v, sem_i).wait()

        for h in range(n_halves):
            def enq(step, _, h=h):
                base = h * HALF + step * ENQ
                for g in range(ENQ // L):
                    v = idx_v[pl.ds(base + g * L, L)]
                    for jj in range(L):
                        slot = step * ENQ + g * L + jj
                        pltpu.async_copy(
                            tab_hbm.at[pl.ds(v[jj], 1), :],
                            rows_v.at[pl.ds(slot, 1), :],
                            sem_g,
                        )
                return _

            lax.fori_loop(0, HALF // ENQ, enq, 0)
            pltpu.make_async_copy(
                tab_hbm.at[pl.ds(0, HALF), :], rows_v, sem_g
            ).wait()

            if h > 0:
                pltpu.make_async_copy(
                    out_v, out_hbm.at[pl.ds(0, HALF // 2)], sem_w
                ).wait()

            def pack(step, _):
                for s in range(L):
                    r = step * L + s
                    for p in range(2):
                        for q in range(D // L):
                            x = rows_v[2 * r + p, pl.ds(q * L, L)]
                            out_v[r, pl.ds(p * D + q * L, L)] = x
                return _

            lax.fori_loop(0, HALF // 2 // L, pack, 0)

            pltpu.async_copy(
                out_v,
                out_hbm.at[pl.ds(wid * (n_per_w // 2) + h * (HALF // 2),
                                 HALF // 2)],
                sem_w,
            )
        pltpu.make_async_copy(
            out_v, out_hbm.at[pl.ds(0, HALF // 2)], sem_w
        ).wait()

    return k(table, idx2)


def kernel(obj_category, sub_category, word_embs):
    (B,) = obj_category.shape
    V, D = word_embs.shape
    idx = jnp.stack(
        [obj_category.astype(jnp.int32), sub_category.astype(jnp.int32)], axis=1
    ).reshape(2 * B)
    return _gather_concat(word_embs, idx.reshape(NW, 2 * B // NW), B, D)
